# baseline (speedup 1.0000x reference)
import functools

import jax
import jax.numpy as jnp
from jax import lax
from jax.experimental import pallas as pl
from jax.experimental.pallas import tpu as pltpu
from jax.experimental.pallas import tpu_sc as plsc

NC, NS, L = 2, 16, 16
NW = NC * NS
B, T, H = 4096, 200, 64
HP = 128
VOCAB = 1000000
SEQ_PER_W = B // NW
CH = 1
NIT = SEQ_PER_W // CH
ROWS = CH * T
NBUF = 3
ROWS_W = SEQ_PER_W * T


def _body(ids_hbm, tok_hbm, pos_hbm, out_hbm, idx_v, gbuf_v, obuf_v, pos_v, *sems):
    gsems, wsems = sems[:NBUF], sems[NBUF:]
    wid = lax.axis_index("s") * NC + lax.axis_index("c")
    row_base = wid * ROWS_W
    pltpu.sync_copy(ids_hbm.at[pl.ds(row_base, ROWS_W)], idx_v)
    pltpu.sync_copy(pos_hbm, pos_v)

    def start_gather(c, b):
        pltpu.async_copy(
            tok_hbm.at[idx_v.at[pl.ds(c * ROWS, ROWS)]], gbuf_v.at[b], gsems[b]
        )

    def wait_gather(c, b):
        pltpu.make_async_copy(
            tok_hbm.at[idx_v.at[pl.ds(c * ROWS, ROWS)]], gbuf_v.at[b], gsems[b]
        ).wait()

    for b in range(2):
        start_gather(b, b)

    @pl.loop(0, 132, step=6)
    def _grp(g):
        for bb in range(6):
            b = bb % NBUF
            bo = bb % 2
            c = g + bb

            @pl.when(c < NIT)
            def _chunk():
                wait_gather(c, b)

                nxt = c + 2

                @pl.when(nxt < NIT)
                def _prefetch():
                    start_gather(nxt, (b + 2) % NBUF)

                @pl.when(c >= 2)
                def _drain():
                    pltpu.make_async_copy(
                        obuf_v.at[bo], out_hbm.at[pl.ds(0, ROWS)], wsems[bo]
                    ).wait()

                @pl.loop(0, T, unroll=2)
                def _row(t):
                    for cc in range(H // L):
                        sl = pl.ds(cc * L, L)
                        obuf_v[bo, t, sl] = gbuf_v[b, t, sl] + pos_v[t, sl]

                row0 = row_base + c * ROWS
                pltpu.async_copy(
                    obuf_v.at[bo], out_hbm.at[pl.ds(row0, ROWS)], wsems[bo]
                )

    for bo in range(2):
        pltpu.make_async_copy(
            obuf_v.at[bo], out_hbm.at[pl.ds(0, ROWS)], wsems[bo]
        ).wait()


@jax.jit
def _run(ids_flat, token_emb, pos_emb):
    mesh = plsc.VectorSubcoreMesh(
        core_axis_name="c", subcore_axis_name="s", num_cores=NC, num_subcores=NS
    )
    k = pl.kernel(
        _body,
        out_type=jax.ShapeDtypeStruct((B * T, HP), jnp.float32),
        mesh=mesh,
        compiler_params=pltpu.CompilerParams(use_tc_tiling_on_sc=False),
        scratch_types=[
            pltpu.VMEM((ROWS_W,), jnp.int32),
            pltpu.VMEM((NBUF, ROWS, H), jnp.float32),
            pltpu.VMEM((2, ROWS, HP), jnp.float32),
            pltpu.VMEM((T, H), jnp.float32),
        ]
        + [pltpu.SemaphoreType.DMA] * (NBUF + 2),
    )
    return k(ids_flat, token_emb, pos_emb)


def kernel(input_ids, token_emb, pos_emb):
    ids_flat = input_ids.reshape(B * T).astype(jnp.int32)
    out = _run(ids_flat, token_emb, pos_emb)
    return out.reshape(B, T, HP)[:, :, :H]


# speedup vs baseline: 1.1015x; 1.1015x over previous
import functools

import jax
import jax.numpy as jnp
from jax import lax
from jax.experimental import pallas as pl
from jax.experimental.pallas import tpu as pltpu
from jax.experimental.pallas import tpu_sc as plsc

NC, NS, L = 2, 16, 16
NW = NC * NS
B, T, H = 4096, 200, 64
HP = 128
VOCAB = 1000000
SEQ_PER_W = B // NW
CH = 1
NIT = SEQ_PER_W // CH
ROWS = CH * T
NBUF = 3
ROWS_W = SEQ_PER_W * T


def _body(ids_hbm, tok_hbm, pos_hbm, out_hbm, idx_v, rows_v, pos_v, *sems):
    gsems, wsems = sems[:NBUF], sems[NBUF:]
    wid = lax.axis_index("s") * NC + lax.axis_index("c")
    row_base = wid * ROWS_W
    pltpu.sync_copy(ids_hbm.at[pl.ds(row_base, ROWS_W)], idx_v)
    pltpu.sync_copy(pos_hbm, pos_v)

    def start_gather(c, b):
        pltpu.async_copy(
            tok_hbm.at[idx_v.at[pl.ds(c * ROWS, ROWS)]], rows_v.at[b], gsems[b]
        )

    def wait_gather(c, b):
        pltpu.make_async_copy(
            tok_hbm.at[idx_v.at[pl.ds(c * ROWS, ROWS)]], rows_v.at[b], gsems[b]
        ).wait()

    for b in range(2):
        start_gather(b, b)

    @pl.loop(0, NIT + 1, step=NBUF)
    def _grp(g):
        for b in range(NBUF):
            c = g + b

            @pl.when(c < NIT)
            def _chunk():
                wait_gather(c, b)

                @pl.loop(0, T, unroll=2)
                def _row(t):
                    for cc in range(H // L):
                        sl = pl.ds(cc * L, L)
                        rows_v[b, t, sl] = rows_v[b, t, sl] + pos_v[t, sl]

                nb = (b + 2) % NBUF
                nxt = c + 2

                @pl.when(nxt < NIT)
                def _prefetch():
                    @pl.when(c >= 1)
                    def _drain():
                        pltpu.make_async_copy(
                            rows_v.at[nb], out_hbm.at[pl.ds(0, ROWS)], wsems[nb]
                        ).wait()

                    start_gather(nxt, nb)

                row0 = row_base + c * ROWS
                pltpu.async_copy(
                    rows_v.at[b], out_hbm.at[pl.ds(row0, ROWS)], wsems[b]
                )

    for b in range(NBUF):
        pltpu.make_async_copy(
            rows_v.at[b], out_hbm.at[pl.ds(0, ROWS)], wsems[b]
        ).wait()


@jax.jit
def _run(ids_flat, tok_padded, pos_emb):
    mesh = plsc.VectorSubcoreMesh(
        core_axis_name="c", subcore_axis_name="s", num_cores=NC, num_subcores=NS
    )
    k = pl.kernel(
        _body,
        out_type=jax.ShapeDtypeStruct((B * T, HP), jnp.float32),
        mesh=mesh,
        compiler_params=pltpu.CompilerParams(use_tc_tiling_on_sc=False),
        scratch_types=[
            pltpu.VMEM((ROWS_W,), jnp.int32),
            pltpu.VMEM((NBUF, ROWS, HP), jnp.float32),
            pltpu.VMEM((T, H), jnp.float32),
        ]
        + [pltpu.SemaphoreType.DMA] * (2 * NBUF),
    )
    return k(ids_flat, tok_padded, pos_emb)


def kernel(input_ids, token_emb, pos_emb):
    ids_flat = input_ids.reshape(B * T).astype(jnp.int32)
    tok_padded = jnp.pad(token_emb, ((0, 0), (0, HP - H)))
    out = _run(ids_flat, tok_padded, pos_emb)
    return out.reshape(B, T, HP)[:, :, :H]
